# baseline (speedup 1.0000x reference)
import functools

import jax
import jax.numpy as jnp
from jax import lax
from jax.experimental import pallas as pl
from jax.experimental.pallas import tpu as pltpu
from jax.experimental.pallas import tpu_sc as plsc

B, S, D = 4, 4096, 1024
NC, NS = 2, 16
NW = NC * NS
ROWS_PER_W = S // NW
C = 16
CF = C * D
NCHUNK = ROWS_PER_W // C

_mesh = plsc.VectorSubcoreMesh(core_axis_name="c", subcore_axis_name="s")


@functools.partial(
    pl.kernel,
    out_type=jax.ShapeDtypeStruct((B, S * D), jnp.float32),
    mesh=_mesh,
    scratch_types=[
        pltpu.VMEM((CF,), jnp.float32),
        pltpu.VMEM((CF,), jnp.float32),
    ],
)
def _sc_add(x_hbm, emb_hbm, out_hbm, ebuf, xbuf):
    cid = lax.axis_index("c")
    sid = lax.axis_index("s")
    wid = sid * NC + cid
    base0 = wid * (ROWS_PER_W * D)

    def do_chunk(k, carry):
        off = base0 + k * CF
        pltpu.sync_copy(emb_hbm.at[pl.ds(off, CF)], ebuf)
        for b in range(B):
            pltpu.sync_copy(x_hbm.at[b, pl.ds(off, CF)], xbuf)

            def add16(i, c2):
                sl = pl.ds(i * 16, 16)
                xbuf[sl] = xbuf[sl] + ebuf[sl]
                return c2

            lax.fori_loop(0, CF // 16, add16, 0)
            pltpu.sync_copy(xbuf, out_hbm.at[b, pl.ds(off, CF)])
        return carry

    lax.fori_loop(0, NCHUNK, do_chunk, 0)


def kernel(x, emb_table):
    xf = x.reshape(B, S * D)
    ef = emb_table.reshape(-1)
    return _sc_add(xf, ef).reshape(B, S, D)


# speedup vs baseline: 1.6668x; 1.6668x over previous
import functools

import jax
import jax.numpy as jnp
from jax import lax
from jax.experimental import pallas as pl
from jax.experimental.pallas import tpu as pltpu
from jax.experimental.pallas import tpu_sc as plsc

B, S, D = 4, 4096, 1024
NC, NS = 2, 16
NW = NC * NS
ROWS_PER_W = S // NW

_mesh = plsc.VectorSubcoreMesh(core_axis_name="c", subcore_axis_name="s")


def _make_sc_add(nb: int, c_rows: int):
    cf = c_rows * D
    nch = ROWS_PER_W // c_rows

    @functools.partial(
        pl.kernel,
        out_type=jax.ShapeDtypeStruct((nb, S * D), jnp.float32),
        mesh=_mesh,
        scratch_types=[
            pltpu.VMEM((2, cf), jnp.float32),
            pltpu.VMEM((nb, 2, cf), jnp.float32),
            pltpu.SemaphoreType.DMA,
            pltpu.SemaphoreType.DMA,
            pltpu.SemaphoreType.DMA,
            pltpu.SemaphoreType.DMA,
        ],
    )
    def sc_add(x_hbm, emb_hbm, out_hbm, ebuf, xbuf, isem0, isem1, osem0, osem1):
        cid = lax.axis_index("c")
        sid = lax.axis_index("s")
        wid = sid * NC + cid
        base0 = wid * (ROWS_PER_W * D)
        isems = (isem0, isem1)
        osems = (osem0, osem1)

        def in_cps(k, p):
            off = base0 + k * cf
            cps = [pltpu.make_async_copy(
                emb_hbm.at[pl.ds(off, cf)], ebuf.at[p], isems[p])]
            for b in range(nb):
                cps.append(pltpu.make_async_copy(
                    x_hbm.at[b, pl.ds(off, cf)], xbuf.at[b, p], isems[p]))
            return cps

        def out_cps(k, p):
            off = base0 + k * cf
            return [pltpu.make_async_copy(
                xbuf.at[b, p], out_hbm.at[b, pl.ds(off, cf)], osems[p])
                for b in range(nb)]

        def compute(p):
            @plsc.parallel_loop(0, cf // 16, unroll=8)
            def _(i):
                sl = pl.ds(i * 16, 16)
                e = ebuf[p, sl]
                for b in range(nb):
                    xbuf[b, p, sl] = xbuf[b, p, sl] + e

        for cp in in_cps(0, 0):
            cp.start()
        for k in range(nch):
            p = k & 1
            if k + 1 < nch:
                for cp in in_cps(k + 1, 1 - p):
                    cp.start()
            for cp in in_cps(k, p):
                cp.wait()
            if k >= 2:
                for cp in out_cps(k - 2, p):
                    cp.wait()
            compute(p)
            for cp in out_cps(k, p):
                cp.start()
        for k in (nch - 2, nch - 1):
            if k >= 0:
                for cp in out_cps(k, k & 1):
                    cp.wait()

    return sc_add


_sc_add_full = _make_sc_add(B, 8)


def kernel(x, emb_table):
    xf = x.reshape(B, S * D)
    ef = emb_table.reshape(-1)
    return _sc_add_full(xf, ef).reshape(B, S, D)


# speedup vs baseline: 4.8571x; 2.9140x over previous
import functools

import jax
import jax.numpy as jnp
from jax import lax
from jax.experimental import pallas as pl
from jax.experimental.pallas import tpu as pltpu
from jax.experimental.pallas import tpu_sc as plsc

B, S, D = 4, 4096, 1024
NC, NS = 2, 16
NW = NC * NS
ROWS_PER_W = S // NW
LANES = 16
DV = D // LANES

_mesh = plsc.VectorSubcoreMesh(core_axis_name="c", subcore_axis_name="s")


def _make_sc_add(nb: int, c_rows: int):
    cf = c_rows * D
    nch = ROWS_PER_W // c_rows
    shift = DV.bit_length() - 1

    @functools.partial(
        pl.kernel,
        out_type=jax.ShapeDtypeStruct((nb, S, D), jnp.float32),
        mesh=_mesh,
        scratch_types=[
            pltpu.VMEM((2, c_rows, D), jnp.float32),
            pltpu.VMEM((nb, 2, c_rows, D), jnp.float32),
            pltpu.SemaphoreType.DMA,
            pltpu.SemaphoreType.DMA,
            pltpu.SemaphoreType.DMA,
            pltpu.SemaphoreType.DMA,
        ],
    )
    def sc_add(x_hbm, emb_hbm, out_hbm, ebuf, xbuf, isem0, isem1, osem0, osem1):
        cid = lax.axis_index("c")
        sid = lax.axis_index("s")
        wid = sid * NC + cid
        row_base = wid * ROWS_PER_W
        isems = (isem0, isem1)
        osems = (osem0, osem1)

        def in_cps(k, p):
            r0 = row_base + k * c_rows
            cps = [pltpu.make_async_copy(
                emb_hbm.at[pl.ds(r0, c_rows)], ebuf.at[p], isems[p])]
            for b in range(nb):
                cps.append(pltpu.make_async_copy(
                    x_hbm.at[b, pl.ds(r0, c_rows)], xbuf.at[b, p], isems[p]))
            return cps

        def out_cps(k, p):
            r0 = row_base + k * c_rows
            return [pltpu.make_async_copy(
                xbuf.at[b, p], out_hbm.at[b, pl.ds(r0, c_rows)], osems[p])
                for b in range(nb)]

        def compute(p):
            @plsc.parallel_loop(0, c_rows * DV, unroll=8)
            def _(i):
                r = lax.shift_right_logical(i, shift)
                sl = pl.ds((i & (DV - 1)) * LANES, LANES)
                e = ebuf[p, r, sl]
                for b in range(nb):
                    xbuf[b, p, r, sl] = xbuf[b, p, r, sl] + e

        for cp in in_cps(0, 0):
            cp.start()
        for k in range(nch):
            p = k & 1
            if k + 1 < nch:
                for cp in in_cps(k + 1, 1 - p):
                    cp.start()
            for cp in in_cps(k, p):
                cp.wait()
            if k >= 2:
                for cp in out_cps(k - 2, p):
                    cp.wait()
            compute(p)
            for cp in out_cps(k, p):
                cp.start()
        for k in (nch - 2, nch - 1):
            if k >= 0:
                for cp in out_cps(k, k & 1):
                    cp.wait()

    return sc_add


_sc_add_full = _make_sc_add(B, 8)


def kernel(x, emb_table):
    return _sc_add_full(x, emb_table)
